# Initial kernel scaffold; baseline (speedup 1.0000x reference)
import jax
import jax.numpy as jnp
from jax.experimental import pallas as pl


def kernel(input_ids, emb_table, fc_w, fc_b):
    raise NotImplementedError("write your pallas kernel here")


# speedup vs baseline: 7.9258x; 7.9258x over previous
import functools

import jax
import jax.numpy as jnp
from jax import lax
from jax.experimental import pallas as pl
from jax.experimental.pallas import tpu as pltpu
from jax.experimental.pallas import tpu_sc as plsc

_NC = 2
_NS = 16
_NW = _NC * _NS
_LANES = 16


def _make_pool_kernel(B, L, V, D):
    bpw = B // _NW
    rows_per_chunk = 2
    idx_per_chunk = rows_per_chunk * L
    ch = bpw // rows_per_chunk
    inv_l = 1.0 / L
    nj = D // _LANES

    mesh = plsc.VectorSubcoreMesh(core_axis_name="c", subcore_axis_name="s")

    @functools.partial(
        pl.kernel,
        mesh=mesh,
        out_type=jax.ShapeDtypeStruct((B * D,), jnp.float32),
        scratch_types=[
            pltpu.VMEM((ch, idx_per_chunk), jnp.int32),
            pltpu.VMEM((idx_per_chunk, D), jnp.float32),
            pltpu.VMEM((idx_per_chunk, D), jnp.float32),
            pltpu.VMEM((bpw * D,), jnp.float32),
            pltpu.SemaphoreType.DMA,
            pltpu.SemaphoreType.DMA,
        ],
    )
    def pool(ids_hbm, table_hbm, out_hbm, idx_v, rows_a, rows_b, pooled_v,
             sem_a, sem_b):
        wid = lax.axis_index("s") * _NC + lax.axis_index("c")
        bufs = (rows_a, rows_b)
        sems = (sem_a, sem_b)

        pltpu.sync_copy(ids_hbm.at[wid], idx_v)

        pltpu.make_async_copy(
            table_hbm.at[idx_v.at[0]], rows_a, sem_a).start()

        def step(i, carry):
            g0 = i * 2
            for b in range(2):
                g = g0 + b
                nb = 1 - b

                @pl.when(g + 1 < ch)
                def _fire():
                    pltpu.make_async_copy(
                        table_hbm.at[idx_v.at[g + 1]], bufs[nb],
                        sems[nb]).start()

                pltpu.make_async_copy(
                    table_hbm.at[idx_v.at[g]], bufs[b], sems[b]).wait()

                buf = bufs[b]
                for r in range(rows_per_chunk):
                    row_base = (g * rows_per_chunk + r) * D
                    for jg in range(0, nj, 4):
                        js = range(jg, jg + 4)
                        accs = {j: buf[r * L, pl.ds(j * _LANES, _LANES)]
                                for j in js}
                        for t in range(1, L):
                            for j in js:
                                accs[j] = accs[j] + buf[
                                    r * L + t, pl.ds(j * _LANES, _LANES)]
                        for j in js:
                            pooled_v[pl.ds(row_base + j * _LANES,
                                           _LANES)] = accs[j] * inv_l
            return carry

        lax.fori_loop(0, ch // 2, step, 0)

        pltpu.sync_copy(pooled_v, out_hbm.at[pl.ds(wid * bpw * D, bpw * D)])

    return pool


def _mm_body(x_ref, w_ref, b_ref, o_ref):
    o_ref[...] = jnp.dot(
        x_ref[...], w_ref[...], preferred_element_type=jnp.float32
    ) + b_ref[...]


@jax.jit
def kernel(input_ids, emb_table, fc_w, fc_b):
    B, L = input_ids.shape
    V, D = emb_table.shape
    O = fc_w.shape[1]
    bpw = B // _NW
    ch = bpw // 2

    ids = input_ids.astype(jnp.int32).reshape(_NW, ch, 2 * L)
    pool = _make_pool_kernel(B, L, V, D)
    pooled = pool(ids, emb_table).reshape(B, D)

    bm = 1024
    out = pl.pallas_call(
        _mm_body,
        grid=(B // bm,),
        in_specs=[
            pl.BlockSpec((bm, D), lambda i: (i, 0)),
            pl.BlockSpec((D, O), lambda i: (0, 0)),
            pl.BlockSpec((1, O), lambda i: (0, 0)),
        ],
        out_specs=pl.BlockSpec((bm, O), lambda i: (i, 0)),
        out_shape=jax.ShapeDtypeStruct((B, O), jnp.float32),
    )(pooled, fc_w, fc_b.reshape(1, O))
    return out


# speedup vs baseline: 11.4254x; 1.4415x over previous
import functools

import jax
import jax.numpy as jnp
from jax import lax
from jax.experimental import pallas as pl
from jax.experimental.pallas import tpu as pltpu
from jax.experimental.pallas import tpu_sc as plsc

_NC = 2
_NS = 16
_NW = _NC * _NS


def _make_pool(B, L, V, D):
    bpw = B // _NW
    toks = bpw * L
    cw = 128
    ch = toks // cw
    nsec = 2
    chs = ch // nsec

    mesh = plsc.VectorSubcoreMesh(core_axis_name="c", subcore_axis_name="s")

    @functools.partial(
        pl.kernel,
        mesh=mesh,
        out_type=jax.ShapeDtypeStruct((B, D), jnp.float32),
        scratch_types=[
            pltpu.VMEM((chs, cw), jnp.int32),
            pltpu.VMEM((chs, cw), jnp.int32),
            pltpu.VMEM((cw, D), jnp.float32),
            pltpu.VMEM((cw, D), jnp.float32),
            pltpu.VMEM_SHARED((_NS * bpw, D), jnp.float32),
            pltpu.SemaphoreType.DMA,
            pltpu.SemaphoreType.DMA,
            pltpu.SemaphoreType.DMA,
            pltpu.SemaphoreType.DMA,
        ],
    )
    def pool(ids_hbm, table_hbm, dst_hbm, zeros_hbm, out_hbm,
             idx_v, dst_v, rows_a, rows_b, pooled_s,
             gsem_a, gsem_b, ssem_a, ssem_b):
        cid = lax.axis_index("c")
        sid = lax.axis_index("s")
        wid = sid * _NC + cid
        bufs = (rows_a, rows_b)
        gsems = (gsem_a, gsem_b)
        ssems = (ssem_a, ssem_b)

        pltpu.sync_copy(zeros_hbm, pooled_s.at[pl.ds(sid * bpw, bpw)])

        for sec in range(nsec):
            pltpu.sync_copy(ids_hbm.at[wid, sec], idx_v)
            pltpu.sync_copy(dst_hbm.at[sid, sec], dst_v)

            pltpu.make_async_copy(
                table_hbm.at[idx_v.at[0]], rows_a, gsem_a).start()

            def step(i, carry):
                g0 = i * 2
                for b in range(2):
                    g = g0 + b
                    nb = 1 - b

                    @pl.when(g + 1 < chs)
                    def _fire():
                        pltpu.make_async_copy(
                            table_hbm.at[idx_v.at[g + 1]], bufs[nb],
                            gsems[nb]).start()

                    pltpu.make_async_copy(
                        table_hbm.at[idx_v.at[g]], bufs[b], gsems[b]).wait()

                    pltpu.async_copy(
                        bufs[b], pooled_s.at[dst_v.at[g]], ssems[b],
                        add=True)
                    pltpu.make_async_copy(
                        bufs[b], pooled_s.at[dst_v.at[g]], ssems[b]).wait()
                return carry

            lax.fori_loop(0, chs // 2, step, 0)

        pltpu.sync_copy(pooled_s.at[pl.ds(sid * bpw, bpw)],
                        out_hbm.at[pl.ds(wid * bpw, bpw)])

    return pool


def _mm_body(x_ref, w_ref, b_ref, o_ref):
    o_ref[...] = jnp.dot(
        x_ref[...], w_ref[...], preferred_element_type=jnp.float32
    ) + b_ref[...]


@jax.jit
def kernel(input_ids, emb_table, fc_w, fc_b):
    B, L = input_ids.shape
    V, D = emb_table.shape
    O = fc_w.shape[1]
    bpw = B // _NW
    toks = bpw * L
    cw = 128
    ch = toks // cw

    nsec = 2
    chs = ch // nsec
    ids = input_ids.astype(jnp.int32).reshape(_NW, nsec, chs, cw)
    local = jnp.arange(toks, dtype=jnp.int32) // L
    dst = (jnp.arange(_NS, dtype=jnp.int32)[:, None] * bpw
           + local[None, :]).reshape(_NS, nsec, chs, cw)
    zeros = jnp.zeros((bpw, D), jnp.float32)

    pool = _make_pool(B, L, V, D)
    pooled = pool(ids, emb_table, dst, zeros)

    w_scaled = fc_w * (1.0 / L)

    bm = 1024
    out = pl.pallas_call(
        _mm_body,
        grid=(B // bm,),
        in_specs=[
            pl.BlockSpec((bm, D), lambda i: (i, 0)),
            pl.BlockSpec((D, O), lambda i: (0, 0)),
            pl.BlockSpec((1, O), lambda i: (0, 0)),
        ],
        out_specs=pl.BlockSpec((bm, O), lambda i: (i, 0)),
        out_shape=jax.ShapeDtypeStruct((B, O), jnp.float32),
    )(pooled, w_scaled, fc_b.reshape(1, O))
    return out


# speedup vs baseline: 11.4367x; 1.0010x over previous
import functools

import jax
import jax.numpy as jnp
from jax import lax
from jax.experimental import pallas as pl
from jax.experimental.pallas import tpu as pltpu
from jax.experimental.pallas import tpu_sc as plsc

_NC = 2
_NS = 16
_NW = _NC * _NS


def _make_pool(B, L, V, D):
    bpw = B // _NW
    toks = bpw * L
    cw = 128
    ch = toks // cw
    nsec = 2
    chs = ch // nsec

    mesh = plsc.VectorSubcoreMesh(core_axis_name="c", subcore_axis_name="s")

    @functools.partial(
        pl.kernel,
        mesh=mesh,
        out_type=jax.ShapeDtypeStruct((B, D), jnp.float32),
        scratch_types=[
            pltpu.VMEM((chs, cw), jnp.int32),
            pltpu.VMEM((chs, cw), jnp.int32),
            pltpu.VMEM((cw, D), jnp.float32),
            pltpu.VMEM((cw, D), jnp.float32),
            pltpu.VMEM_SHARED((_NS * bpw, D), jnp.float32),
            pltpu.SemaphoreType.DMA,
            pltpu.SemaphoreType.DMA,
            pltpu.SemaphoreType.DMA,
            pltpu.SemaphoreType.DMA,
        ],
    )
    def pool(ids_hbm, table_hbm, dst_hbm, zeros_hbm, out_hbm,
             idx_v, dst_v, rows_a, rows_b, pooled_s,
             gsem_a, gsem_b, ssem_a, ssem_b):
        cid = lax.axis_index("c")
        sid = lax.axis_index("s")
        wid = sid * _NC + cid
        bufs = (rows_a, rows_b)
        gsems = (gsem_a, gsem_b)
        ssems = (ssem_a, ssem_b)

        pltpu.sync_copy(zeros_hbm, pooled_s.at[pl.ds(sid * bpw, bpw)])

        for sec in range(nsec):
            pltpu.sync_copy(ids_hbm.at[wid, sec], idx_v)
            pltpu.sync_copy(dst_hbm.at[sid, sec], dst_v)

            pltpu.make_async_copy(
                table_hbm.at[idx_v.at[0]], rows_a, gsem_a).start()

            def step(i, carry):
                g0 = i * 2
                for b in range(2):
                    g = g0 + b
                    nb = 1 - b

                    @pl.when(g >= 1)
                    def _drain():
                        pltpu.make_async_copy(
                            bufs[nb], pooled_s.at[dst_v.at[g - 1]],
                            ssems[nb]).wait()

                    @pl.when(g + 1 < chs)
                    def _fire():
                        pltpu.make_async_copy(
                            table_hbm.at[idx_v.at[g + 1]], bufs[nb],
                            gsems[nb]).start()

                    pltpu.make_async_copy(
                        table_hbm.at[idx_v.at[g]], bufs[b], gsems[b]).wait()

                    pltpu.async_copy(
                        bufs[b], pooled_s.at[dst_v.at[g]], ssems[b],
                        add=True)
                return carry

            lax.fori_loop(0, chs // 2, step, 0)

            pltpu.make_async_copy(
                bufs[(chs - 1) % 2], pooled_s.at[dst_v.at[chs - 1]],
                ssems[(chs - 1) % 2]).wait()

        pltpu.sync_copy(pooled_s.at[pl.ds(sid * bpw, bpw)],
                        out_hbm.at[pl.ds(wid * bpw, bpw)])

    return pool


def _mm_body(x_ref, w_ref, b_ref, o_ref):
    o_ref[...] = jnp.dot(
        x_ref[...], w_ref[...], preferred_element_type=jnp.float32
    ) + b_ref[...]


@jax.jit
def kernel(input_ids, emb_table, fc_w, fc_b):
    B, L = input_ids.shape
    V, D = emb_table.shape
    O = fc_w.shape[1]
    bpw = B // _NW
    toks = bpw * L
    cw = 128
    ch = toks // cw

    nsec = 2
    chs = ch // nsec
    ids = input_ids.astype(jnp.int32).reshape(_NW, nsec, chs, cw)
    local = jnp.arange(toks, dtype=jnp.int32) // L
    dst = (jnp.arange(_NS, dtype=jnp.int32)[:, None] * bpw
           + local[None, :]).reshape(_NS, nsec, chs, cw)
    zeros = jnp.zeros((bpw, D), jnp.float32)

    pool = _make_pool(B, L, V, D)
    pooled = pool(ids, emb_table, dst, zeros)

    w_scaled = fc_w * (1.0 / L)

    bm = 1024
    out = pl.pallas_call(
        _mm_body,
        grid=(B // bm,),
        in_specs=[
            pl.BlockSpec((bm, D), lambda i: (i, 0)),
            pl.BlockSpec((D, O), lambda i: (0, 0)),
            pl.BlockSpec((1, O), lambda i: (0, 0)),
        ],
        out_specs=pl.BlockSpec((bm, O), lambda i: (i, 0)),
        out_shape=jax.ShapeDtypeStruct((B, O), jnp.float32),
    )(pooled, w_scaled, fc_b.reshape(1, O))
    return out
